# baseline (speedup 1.0000x reference)
import jax
import jax.numpy as jnp
from jax.experimental import pallas as pl
from jax.experimental.pallas import tpu as pltpu

_B, _N, _D = 32, 20, 2048
_K = 4
_NP = 24
_TO = 512


def _select(x2, p_ref):
    blocks = []
    for b in range(_B):
        xb = x2[b * _N:(b + 1) * _N]
        g = jax.lax.dot_general(xb, xb, (((1,), (1,)), ((), ())),
                                preferred_element_type=jnp.float32)
        sq = jnp.sum(xb * xb, axis=1)
        blocks.append((sq[:, None] + (-2.0 * g)) + sq[None, :])
    adj = jnp.concatenate(blocks, axis=1)
    srow = jax.lax.broadcasted_iota(jnp.int32, (_N, _B * _N), 0)
    cnt = jnp.zeros((_N, _B * _N), jnp.int32)
    for mp in range(_N):
        row = adj[mp:mp + 1, :]
        hit = (row < adj) | ((row == adj) & (mp < srow))
        cnt = cnt + hit.astype(jnp.int32)
    for b in range(_B):
        cb = cnt[:, b * _N:(b + 1) * _N]
        for j in range(_K):
            p_ref[b, :, _NP * j:_NP * j + _N] = (cb == j).astype(jnp.float32)


def _body(x2_hbm, w_ref, bias_ref, out_ref, x2_vmem, xb_vmem, p_ref, sem):
    @pl.when(pl.program_id(0) == 0)
    def _():
        copy = pltpu.make_async_copy(x2_hbm, x2_vmem, sem)
        copy.start()
        copy.wait()
        x2f = x2_vmem[...]
        xb_vmem[...] = x2f.astype(jnp.bfloat16)
        _select(x2f, p_ref)

    x2b = xb_vmem[...]
    w = w_ref[...]
    wab = w[:, :_D].astype(jnp.bfloat16)
    wbb = w[:, _D:].astype(jnp.bfloat16)
    p1 = jax.lax.dot_general(x2b, wab, (((1,), (1,)), ((), ())),
                             preferred_element_type=jnp.float32)
    v = jax.lax.dot_general(x2b, wbb, (((1,), (1,)), ((), ())),
                            preferred_element_type=jnp.float32)
    u = p1 - v
    bias = bias_ref[0]
    for b in range(_B):
        ub = u[b * _N:(b + 1) * _N]
        vb = v[b * _N:(b + 1) * _N]
        pt = p_ref[b]
        gv = jax.lax.dot_general(pt, vb, (((0,), (0,)), ((), ())),
                                 preferred_element_type=jnp.float32)
        m01 = jnp.maximum(gv[0:_N], gv[_NP:_NP + _N])
        m23 = jnp.maximum(gv[2 * _NP:2 * _NP + _N], gv[3 * _NP:3 * _NP + _N])
        maxv = jnp.maximum(m01, m23)
        r = jnp.max(ub + maxv, axis=0)
        out_ref[b, :] = jnp.maximum(r + bias, 0.0)


def kernel(x, W, b):
    x2 = x.reshape(_B * _N, _D)
    out = pl.pallas_call(
        _body,
        grid=(_D // _TO,),
        in_specs=[
            pl.BlockSpec(memory_space=pl.ANY),
            pl.BlockSpec((_TO, 2 * _D), lambda i: (i, 0)),
            pl.BlockSpec((1, _TO), lambda i: (0, i)),
        ],
        out_specs=pl.BlockSpec((_B, _TO), lambda i: (0, i)),
        out_shape=jax.ShapeDtypeStruct((_B, _D), jnp.float32),
        scratch_shapes=[
            pltpu.VMEM((_B * _N, _D), jnp.float32),
            pltpu.VMEM((_B * _N, _D), jnp.bfloat16),
            pltpu.VMEM((_B, _N, _K * _NP), jnp.float32),
            pltpu.SemaphoreType.DMA,
        ],
    )(x2, W, b.reshape(1, _D))
    return out.reshape(_B, _D, 1, 1)


# speedup vs baseline: 1.1263x; 1.1263x over previous
import jax
import jax.numpy as jnp
from jax.experimental import pallas as pl
from jax.experimental.pallas import tpu as pltpu

_B, _N, _D = 32, 20, 2048
_K = 4
_NP = 24
_TO = 512


def _select(x3, p_ref):
    blocks = []
    for b in range(_B):
        xb = x3[b]
        g = jax.lax.dot_general(xb, xb, (((1,), (1,)), ((), ())),
                                preferred_element_type=jnp.float32)
        sq = jnp.sum(xb * xb, axis=1)
        blocks.append((sq[:, None] + (-2.0 * g)) + sq[None, :])
    adj = jnp.concatenate(blocks, axis=1)
    srow = jax.lax.broadcasted_iota(jnp.int32, (_N, _B * _N), 0)
    cnt = jnp.zeros((_N, _B * _N), jnp.int32)
    for mp in range(_N):
        row = adj[mp:mp + 1, :]
        hit = (row < adj) | ((row == adj) & (mp < srow))
        cnt = cnt + hit.astype(jnp.int32)
    for b in range(_B):
        cb = cnt[:, b * _N:(b + 1) * _N]
        for j in range(_K):
            p_ref[b, :, _NP * j:_NP * j + _N] = (cb == j).astype(jnp.float32)


def _body(x_hbm, w_ref, bias_ref, out_ref, x3_vmem, xb_vmem, p_ref, sem):
    @pl.when(pl.program_id(0) == 0)
    def _():
        copy = pltpu.make_async_copy(x_hbm, x3_vmem, sem)
        copy.start()
        copy.wait()
        x3 = x3_vmem[...]
        xb_vmem[...] = jnp.zeros((_B * _NP, _D), jnp.bfloat16)
        for b in range(_B):
            xb_vmem[b * _NP:b * _NP + _N, :] = x3[b].astype(jnp.bfloat16)
        _select(x3, p_ref)

    x2b = xb_vmem[...]
    w = w_ref[...]
    wab = w[:, :_D].astype(jnp.bfloat16)
    wbb = w[:, _D:].astype(jnp.bfloat16)
    p1 = jax.lax.dot_general(x2b, wab, (((1,), (1,)), ((), ())),
                             preferred_element_type=jnp.float32)
    v = jax.lax.dot_general(x2b, wbb, (((1,), (1,)), ((), ())),
                            preferred_element_type=jnp.float32)
    u = p1 - v
    bias = bias_ref[0]
    for b in range(_B):
        ub = u[b * _NP:b * _NP + _N]
        vb = v[b * _NP:b * _NP + _N]
        pt = p_ref[b]
        gv = jax.lax.dot_general(pt, vb, (((0,), (0,)), ((), ())),
                                 preferred_element_type=jnp.float32)
        m01 = jnp.maximum(gv[0:_N], gv[_NP:_NP + _N])
        m23 = jnp.maximum(gv[2 * _NP:2 * _NP + _N], gv[3 * _NP:3 * _NP + _N])
        maxv = jnp.maximum(m01, m23)
        r = jnp.max(ub + maxv, axis=0)
        out_ref[b, :] = jnp.maximum(r + bias, 0.0)


def kernel(x, W, b):
    out = pl.pallas_call(
        _body,
        grid=(_D // _TO,),
        in_specs=[
            pl.BlockSpec(memory_space=pl.ANY),
            pl.BlockSpec((_TO, 2 * _D), lambda i: (i, 0)),
            pl.BlockSpec((1, _TO), lambda i: (0, i)),
        ],
        out_specs=pl.BlockSpec((_B, _TO), lambda i: (0, i)),
        out_shape=jax.ShapeDtypeStruct((_B, _D), jnp.float32),
        scratch_shapes=[
            pltpu.VMEM((_B, _N, _D), jnp.float32),
            pltpu.VMEM((_B * _NP, _D), jnp.bfloat16),
            pltpu.VMEM((_B, _N, _K * _NP), jnp.float32),
            pltpu.SemaphoreType.DMA,
        ],
    )(x, W, b.reshape(1, _D))
    return out.reshape(_B, _D, 1, 1)
